# Initial kernel scaffold; baseline (speedup 1.0000x reference)
import jax
import jax.numpy as jnp
from jax.experimental import pallas as pl


def kernel(x, emb, fm_w, fm_b, w1, b1, w2, b2, w_last):
    raise NotImplementedError("write your pallas kernel here")


# speedup vs baseline: 37.8405x; 37.8405x over previous
import functools

import jax
import jax.numpy as jnp
from jax import lax
from jax.experimental import pallas as pl
from jax.experimental.pallas import tpu as pltpu
from jax.experimental.pallas import tpu_sc as plsc

B, F, V, D = 4096, 26, 1000, 16
H1, H2 = 400, 400
NC, NS = 2, 16
NW = NC * NS
BF = B * F
PER_W = BF // NW

@functools.cache
def _make_sc_gather():
    mesh = plsc.VectorSubcoreMesh(
        core_axis_name="c", subcore_axis_name="s",
        num_cores=NC, num_subcores=NS)

    @functools.partial(
        pl.kernel,
        mesh=mesh,
        out_type=(
            jax.ShapeDtypeStruct((BF, D), jnp.float32),
            jax.ShapeDtypeStruct((BF,), jnp.float32),
        ),
        scratch_types=[
            pltpu.VMEM((PER_W,), jnp.int32),
            pltpu.VMEM((PER_W, D), jnp.float32),
            pltpu.VMEM((PER_W,), jnp.float32),
            pltpu.VMEM((F * V,), jnp.float32),
            pltpu.SemaphoreType.DMA,
        ],
        compiler_params=pltpu.CompilerParams(
            use_tc_tiling_on_sc=False, needs_layout_passes=False),
    )
    def sc_gather(idx_hbm, emb_hbm, fmw_hbm, dense_out, fmv_out,
                  idx_v, rows_v, fmv_v, fmtab_v, sem_e):
        wid = lax.axis_index("s") * NC + lax.axis_index("c")
        base = wid * PER_W
        pltpu.sync_copy(idx_hbm.at[pl.ds(base, PER_W)], idx_v)
        cp_e = pltpu.async_copy(emb_hbm.at[idx_v], rows_v, sem_e)
        pltpu.sync_copy(fmw_hbm, fmtab_v)

        def fm_body(i, _):
            ii = i * 16
            vals = plsc.load_gather(fmtab_v, [idx_v[pl.ds(ii, 16)]])
            fmv_v[pl.ds(ii, 16)] = vals
            return 0

        lax.fori_loop(0, PER_W // 16, fm_body, 0)
        cp_e.wait()
        pltpu.sync_copy(rows_v, dense_out.at[pl.ds(base, PER_W)])
        pltpu.sync_copy(fmv_v, fmv_out.at[pl.ds(base, PER_W)])

    return sc_gather


BLK = 1024


def _tc_body(dense_ref, fmv_ref, w1_ref, b1_ref, w2_ref, b2_ref, wl_ref,
             fmb_ref, out_ref):
    h = dense_ref[...]
    r = lax.broadcasted_iota(jnp.int32, (F * D, D), 0)
    c = lax.broadcasted_iota(jnp.int32, (F * D, D), 1)
    msel = jnp.where((r % D) == c, 1.0, 0.0).astype(jnp.float32)
    s = jnp.dot(h, msel, preferred_element_type=jnp.float32)
    ss = jnp.dot(h * h, msel, preferred_element_type=jnp.float32)
    inter = 0.5 * jnp.sum(s * s - ss, axis=1)
    lin = jnp.sum(fmv_ref[...], axis=1)
    dn = (((1,), (1,)), ((), ()))
    a1 = lax.dot_general(h, w1_ref[...], dn,
                         preferred_element_type=jnp.float32) + b1_ref[...]
    a1 = jnp.maximum(a1, 0.0)
    a2 = lax.dot_general(a1, w2_ref[...], dn,
                         preferred_element_type=jnp.float32) + b2_ref[...]
    a2 = jnp.maximum(a2, 0.0)
    y = lax.dot_general(a2, wl_ref[...], dn,
                        preferred_element_type=jnp.float32)
    out_ref[...] = lin + inter + y[:, 0] + fmb_ref[0, 0]


def _tc_forward(dense, fmv, w1, b1, w2, b2, wl, fmb):
    return pl.pallas_call(
        _tc_body,
        grid=(B // BLK,),
        in_specs=[
            pl.BlockSpec((BLK, F * D), lambda i: (i, 0)),
            pl.BlockSpec((BLK, F), lambda i: (i, 0)),
            pl.BlockSpec((H1, F * D), lambda i: (0, 0)),
            pl.BlockSpec((1, H1), lambda i: (0, 0)),
            pl.BlockSpec((H2, H1), lambda i: (0, 0)),
            pl.BlockSpec((1, H2), lambda i: (0, 0)),
            pl.BlockSpec((1, H2), lambda i: (0, 0)),
            pl.BlockSpec((1, 1), lambda i: (0, 0)),
        ],
        out_specs=pl.BlockSpec((BLK,), lambda i: (i,)),
        out_shape=jax.ShapeDtypeStruct((B,), jnp.float32),
    )(dense, fmv, w1, b1, w2, b2, wl, fmb)


def kernel(x, emb, fm_w, fm_b, w1, b1, w2, b2, w_last):
    offsets = (jnp.arange(F, dtype=jnp.int32) * V)
    idx = (x + offsets[None, :]).reshape(BF)
    emb_flat = emb.reshape(F * V, D)
    fmw2 = fm_w.reshape(F * V)
    dense_flat, fmv = _make_sc_gather()(idx, emb_flat, fmw2)
    dense = dense_flat.reshape(B, F * D)
    fmv2 = fmv.reshape(B, F)
    return _tc_forward(dense, fmv2, w1, b1.reshape(1, H1), w2,
                       b2.reshape(1, H2), w_last, fm_b.reshape(1, 1))


# speedup vs baseline: 39.1429x; 1.0344x over previous
import functools

import jax
import jax.numpy as jnp
from jax import lax
from jax.experimental import pallas as pl
from jax.experimental.pallas import tpu as pltpu
from jax.experimental.pallas import tpu_sc as plsc

B, F, V, D = 4096, 26, 1000, 16
H1, H2 = 400, 400
NC, NS = 2, 16
NW = NC * NS
BF = B * F
PER_W = BF // NW
ROWS_W = B // NW


@functools.cache
def _make_sc_gather():
    mesh = plsc.VectorSubcoreMesh(
        core_axis_name="c", subcore_axis_name="s",
        num_cores=NC, num_subcores=NS)

    @functools.partial(
        pl.kernel,
        mesh=mesh,
        out_type=(
            jax.ShapeDtypeStruct((B, F * D), jnp.float32),
            jax.ShapeDtypeStruct((B,), jnp.float32),
        ),
        scratch_types=[
            pltpu.VMEM((PER_W,), jnp.int32),
            pltpu.VMEM((PER_W, D), jnp.float32),
            pltpu.VMEM((ROWS_W,), jnp.float32),
            pltpu.VMEM((F * V,), jnp.float32),
            pltpu.SemaphoreType.DMA,
            pltpu.SemaphoreType.DMA,
        ],
        compiler_params=pltpu.CompilerParams(
            use_tc_tiling_on_sc=False, needs_layout_passes=False),
    )
    def sc_gather(idx_hbm, emb_hbm, fmw_hbm, dense_out, lin_out,
                  idx_v, rows_v, lin_v, fmtab_v, sem_e, sem_w):
        wid = lax.axis_index("s") * NC + lax.axis_index("c")
        base = wid * PER_W
        brow = wid * ROWS_W
        pltpu.sync_copy(idx_hbm.at[pl.ds(base, PER_W)], idx_v)
        cp_e = pltpu.async_copy(emb_hbm.at[idx_v], rows_v, sem_e)
        pltpu.sync_copy(fmw_hbm, fmtab_v)

        def fm_body(g, _):
            acc = jnp.zeros((16,), jnp.float32)
            for f in range(F):
                ii = idx_v[pl.ds(f * ROWS_W + g * 16, 16)]
                acc = acc + plsc.load_gather(fmtab_v, [ii])
            lin_v[pl.ds(g * 16, 16)] = acc
            return 0

        lax.fori_loop(0, ROWS_W // 16, fm_body, 0)
        cp_e.wait()
        cps = []
        for f in range(F):
            cps.append(pltpu.async_copy(
                rows_v.at[pl.ds(f * ROWS_W, ROWS_W)],
                dense_out.at[pl.ds(brow, ROWS_W), pl.ds(f * D, D)],
                sem_w))
        for cp in cps:
            cp.wait()
        pltpu.sync_copy(lin_v, lin_out.at[pl.ds(brow, ROWS_W)])

    return sc_gather


BLK = 1024


def _tc_body(dense_ref, lin_ref, w1_ref, b1_ref, w2_ref, b2_ref, wl_ref,
             fmb_ref, out_ref):
    h = dense_ref[...]
    r = lax.broadcasted_iota(jnp.int32, (F * D, D), 0)
    c = lax.broadcasted_iota(jnp.int32, (F * D, D), 1)
    msel = jnp.where((r % D) == c, 1.0, 0.0).astype(jnp.float32)
    s = jnp.dot(h, msel, preferred_element_type=jnp.float32)
    ss = jnp.dot(h * h, msel, preferred_element_type=jnp.float32)
    inter = 0.5 * jnp.sum(s * s - ss, axis=1)
    dn = (((1,), (1,)), ((), ()))
    a1 = lax.dot_general(h, w1_ref[...], dn,
                         preferred_element_type=jnp.float32) + b1_ref[...]
    a1 = jnp.maximum(a1, 0.0)
    a2 = lax.dot_general(a1, w2_ref[...], dn,
                         preferred_element_type=jnp.float32) + b2_ref[...]
    a2 = jnp.maximum(a2, 0.0)
    y = lax.dot_general(a2, wl_ref[...], dn,
                        preferred_element_type=jnp.float32)
    out_ref[...] = lin_ref[...] + inter + y[:, 0] + fmb_ref[0, 0]


def _tc_forward(dense, lin, w1, b1, w2, b2, wl, fmb):
    return pl.pallas_call(
        _tc_body,
        grid=(B // BLK,),
        in_specs=[
            pl.BlockSpec((BLK, F * D), lambda i: (i, 0)),
            pl.BlockSpec((BLK,), lambda i: (i,)),
            pl.BlockSpec((H1, F * D), lambda i: (0, 0)),
            pl.BlockSpec((1, H1), lambda i: (0, 0)),
            pl.BlockSpec((H2, H1), lambda i: (0, 0)),
            pl.BlockSpec((1, H2), lambda i: (0, 0)),
            pl.BlockSpec((1, H2), lambda i: (0, 0)),
            pl.BlockSpec((1, 1), lambda i: (0, 0)),
        ],
        out_specs=pl.BlockSpec((BLK,), lambda i: (i,)),
        out_shape=jax.ShapeDtypeStruct((B,), jnp.float32),
    )(dense, lin, w1, b1, w2, b2, wl, fmb)


def kernel(x, emb, fm_w, fm_b, w1, b1, w2, b2, w_last):
    offsets = (jnp.arange(F, dtype=jnp.int32) * V)
    x_off = x + offsets[None, :]
    idx = x_off.reshape(NW, ROWS_W, F).transpose(0, 2, 1).reshape(BF)
    emb_flat = emb.reshape(F * V, D)
    fmw_flat = fm_w.reshape(F * V)
    dense, lin = _make_sc_gather()(idx, emb_flat, fmw_flat)
    return _tc_forward(dense, lin, w1, b1.reshape(1, H1), w2,
                       b2.reshape(1, H2), w_last, fm_b.reshape(1, 1))


# speedup vs baseline: 39.1778x; 1.0009x over previous
import functools

import jax
import jax.numpy as jnp
from jax import lax
from jax.experimental import pallas as pl
from jax.experimental.pallas import tpu as pltpu
from jax.experimental.pallas import tpu_sc as plsc

B, F, V, D = 4096, 26, 1000, 16
H1, H2 = 400, 400
NC, NS = 2, 16
NW = NC * NS
BF = B * F
PER_W = BF // NW
ROWS_W = B // NW
FD_PAD = 512


@functools.cache
def _make_sc_gather():
    mesh = plsc.VectorSubcoreMesh(
        core_axis_name="c", subcore_axis_name="s",
        num_cores=NC, num_subcores=NS)

    @functools.partial(
        pl.kernel,
        mesh=mesh,
        out_type=(
            jax.ShapeDtypeStruct((B, FD_PAD), jnp.float32),
            jax.ShapeDtypeStruct((B,), jnp.float32),
        ),
        scratch_types=[
            pltpu.VMEM((PER_W,), jnp.int32),
            pltpu.VMEM((PER_W, D), jnp.float32),
            pltpu.VMEM((ROWS_W,), jnp.float32),
            pltpu.VMEM((F * V,), jnp.float32),
            pltpu.SemaphoreType.DMA,
            pltpu.SemaphoreType.DMA,
        ],
        compiler_params=pltpu.CompilerParams(
            use_tc_tiling_on_sc=False, needs_layout_passes=False),
    )
    def sc_gather(idx_hbm, emb_hbm, fmw_hbm, dense_out, lin_out,
                  idx_v, rows_v, lin_v, fmtab_v, sem_e, sem_w):
        wid = lax.axis_index("s") * NC + lax.axis_index("c")
        base = wid * PER_W
        brow = wid * ROWS_W
        pltpu.sync_copy(idx_hbm.at[pl.ds(base, PER_W)], idx_v)
        cp_e = pltpu.async_copy(emb_hbm.at[idx_v], rows_v, sem_e)
        pltpu.sync_copy(fmw_hbm, fmtab_v)

        def fm_body(g, _):
            acc = jnp.zeros((16,), jnp.float32)
            for f in range(F):
                ii = idx_v[pl.ds(f * ROWS_W + g * 16, 16)]
                acc = acc + plsc.load_gather(fmtab_v, [ii])
            lin_v[pl.ds(g * 16, 16)] = acc
            return 0

        lax.fori_loop(0, ROWS_W // 16, fm_body, 0)
        cp_e.wait()
        cps = []
        for f in range(F):
            cps.append(pltpu.async_copy(
                rows_v.at[pl.ds(f * ROWS_W, ROWS_W)],
                dense_out.at[pl.ds(brow, ROWS_W), pl.ds(f * D, D)],
                sem_w))
        for cp in cps:
            cp.wait()
        pltpu.sync_copy(lin_v, lin_out.at[pl.ds(brow, ROWS_W)])

    return sc_gather


BLK = 1024


def _tc_body(dense_ref, lin_ref, w1_ref, b1_ref, w2_ref, b2_ref, wl_ref,
             fmb_ref, out_ref):
    hp = dense_ref[...]
    col = lax.broadcasted_iota(jnp.int32, (BLK, FD_PAD), 1)
    h = jnp.where(col < F * D, hp, 0.0)
    r = lax.broadcasted_iota(jnp.int32, (FD_PAD, D), 0)
    c = lax.broadcasted_iota(jnp.int32, (FD_PAD, D), 1)
    msel = jnp.where((r % D) == c, 1.0, 0.0).astype(jnp.float32)
    s = jnp.dot(h, msel, preferred_element_type=jnp.float32)
    ss = jnp.dot(h * h, msel, preferred_element_type=jnp.float32)
    inter = 0.5 * jnp.sum(s * s - ss, axis=1)
    dn = (((1,), (1,)), ((), ()))
    a1 = lax.dot_general(h, w1_ref[...], dn,
                         preferred_element_type=jnp.float32) + b1_ref[...]
    a1 = jnp.maximum(a1, 0.0)
    a2 = lax.dot_general(a1, w2_ref[...], dn,
                         preferred_element_type=jnp.float32) + b2_ref[...]
    a2 = jnp.maximum(a2, 0.0)
    y = lax.dot_general(a2, wl_ref[...], dn,
                        preferred_element_type=jnp.float32)
    out_ref[...] = lin_ref[...] + inter + y[:, 0] + fmb_ref[0, 0]


def _tc_forward(dense, lin, w1, b1, w2, b2, wl, fmb):
    return pl.pallas_call(
        _tc_body,
        grid=(B // BLK,),
        in_specs=[
            pl.BlockSpec((BLK, FD_PAD), lambda i: (i, 0)),
            pl.BlockSpec((BLK,), lambda i: (i,)),
            pl.BlockSpec((H1, FD_PAD), lambda i: (0, 0)),
            pl.BlockSpec((1, H1), lambda i: (0, 0)),
            pl.BlockSpec((H2, H1), lambda i: (0, 0)),
            pl.BlockSpec((1, H2), lambda i: (0, 0)),
            pl.BlockSpec((1, H2), lambda i: (0, 0)),
            pl.BlockSpec((1, 1), lambda i: (0, 0)),
        ],
        out_specs=pl.BlockSpec((BLK,), lambda i: (i,)),
        out_shape=jax.ShapeDtypeStruct((B,), jnp.float32),
    )(dense, lin, w1, b1, w2, b2, wl, fmb)


def kernel(x, emb, fm_w, fm_b, w1, b1, w2, b2, w_last):
    offsets = (jnp.arange(F, dtype=jnp.int32) * V)
    x_off = x + offsets[None, :]
    idx = x_off.reshape(NW, ROWS_W, F).transpose(0, 2, 1).reshape(BF)
    emb_flat = emb.reshape(F * V, D)
    fmw_flat = fm_w.reshape(F * V)
    dense, lin = _make_sc_gather()(idx, emb_flat, fmw_flat)
    w1p = jnp.pad(w1, ((0, 0), (0, FD_PAD - F * D)))
    return _tc_forward(dense, lin, w1p, b1.reshape(1, H1), w2,
                       b2.reshape(1, H2), w_last, fm_b.reshape(1, 1))


# speedup vs baseline: 45.2987x; 1.1562x over previous
import functools

import jax
import jax.numpy as jnp
from jax import lax
from jax.experimental import pallas as pl
from jax.experimental.pallas import tpu as pltpu
from jax.experimental.pallas import tpu_sc as plsc

B, F, V, D = 4096, 26, 1000, 16
H1, H2 = 400, 400
NC, NS = 2, 16
NW = NC * NS
BF = B * F
PER_W = BF // NW
ROWS_W = B // NW
NRS = 4


@functools.cache
def _make_sc_gather():
    mesh = plsc.VectorSubcoreMesh(
        core_axis_name="c", subcore_axis_name="s",
        num_cores=NC, num_subcores=NS)

    @functools.partial(
        pl.kernel,
        mesh=mesh,
        out_type=(
            jax.ShapeDtypeStruct((NRS, B, 128), jnp.float32),
            jax.ShapeDtypeStruct((B,), jnp.float32),
        ),
        scratch_types=[
            pltpu.VMEM((PER_W,), jnp.int32),
            pltpu.VMEM((PER_W, D), jnp.float32),
            pltpu.VMEM((ROWS_W,), jnp.float32),
            pltpu.VMEM((F * V,), jnp.float32),
            pltpu.SemaphoreType.DMA,
            pltpu.SemaphoreType.DMA,
        ],
        compiler_params=pltpu.CompilerParams(
            use_tc_tiling_on_sc=False, needs_layout_passes=False),
    )
    def sc_gather(idx_hbm, emb_hbm, fmw_hbm, dense_out, lin_out,
                  idx_v, rows_v, lin_v, fmtab_v, sem_e, sem_w):
        wid = lax.axis_index("s") * NC + lax.axis_index("c")
        base = wid * PER_W
        brow = wid * ROWS_W
        pltpu.sync_copy(idx_hbm.at[pl.ds(base, PER_W)], idx_v)
        cp_e = pltpu.async_copy(emb_hbm.at[idx_v], rows_v, sem_e)
        pltpu.sync_copy(fmw_hbm, fmtab_v)

        def fm_body(g, _):
            acc = jnp.zeros((16,), jnp.float32)
            for f in range(F):
                ii = idx_v[pl.ds(f * ROWS_W + g * 16, 16)]
                acc = acc + plsc.load_gather(fmtab_v, [ii])
            lin_v[pl.ds(g * 16, 16)] = acc
            return 0

        lax.fori_loop(0, ROWS_W // 16, fm_body, 0)
        cp_e.wait()
        cps = []
        for f in range(F):
            rs, c0 = (f * D) // 128, (f * D) % 128
            cps.append(pltpu.async_copy(
                rows_v.at[pl.ds(f * ROWS_W, ROWS_W)],
                dense_out.at[rs, pl.ds(brow, ROWS_W), pl.ds(c0, D)],
                sem_w))
        for cp in cps:
            cp.wait()
        pltpu.sync_copy(lin_v, lin_out.at[pl.ds(brow, ROWS_W)])

    return sc_gather


BLK = 1024


def _tc_body(dense_ref, lin_ref, w1_ref, b1_ref, w2_ref, b2_ref, wl_ref,
             fmb_ref, out_ref):
    d4 = [dense_ref[rs] for rs in range(NRS)]
    padcol = lax.broadcasted_iota(jnp.int32, (BLK, 128), 1)
    d4[3] = jnp.where(padcol < (F * D - 3 * 128), d4[3], 0.0)
    r = lax.broadcasted_iota(jnp.int32, (128, D), 0)
    c = lax.broadcasted_iota(jnp.int32, (128, D), 1)
    msel = jnp.where((r % D) == c, 1.0, 0.0).astype(jnp.float32)
    dsum = d4[0] + d4[1] + d4[2] + d4[3]
    dsq = d4[0] * d4[0] + d4[1] * d4[1] + d4[2] * d4[2] + d4[3] * d4[3]
    s = jnp.dot(dsum, msel, preferred_element_type=jnp.float32)
    ss = jnp.dot(dsq, msel, preferred_element_type=jnp.float32)
    inter = 0.5 * jnp.sum(s * s - ss, axis=1)
    dn = (((1,), (1,)), ((), ()))
    a1 = b1_ref[...]
    for rs in range(NRS):
        a1 = a1 + lax.dot_general(d4[rs], w1_ref[:, pl.ds(rs * 128, 128)],
                                  dn, preferred_element_type=jnp.float32)
    a1 = jnp.maximum(a1, 0.0)
    a2 = lax.dot_general(a1, w2_ref[...], dn,
                         preferred_element_type=jnp.float32) + b2_ref[...]
    a2 = jnp.maximum(a2, 0.0)
    y = lax.dot_general(a2, wl_ref[...], dn,
                        preferred_element_type=jnp.float32)
    out_ref[...] = lin_ref[...] + inter + y[:, 0] + fmb_ref[0, 0]


def _tc_forward(dense4, lin, w1, b1, w2, b2, wl, fmb):
    return pl.pallas_call(
        _tc_body,
        grid=(B // BLK,),
        in_specs=[
            pl.BlockSpec((NRS, BLK, 128), lambda i: (0, i, 0)),
            pl.BlockSpec((BLK,), lambda i: (i,)),
            pl.BlockSpec((H1, NRS * 128), lambda i: (0, 0)),
            pl.BlockSpec((1, H1), lambda i: (0, 0)),
            pl.BlockSpec((H2, H1), lambda i: (0, 0)),
            pl.BlockSpec((1, H2), lambda i: (0, 0)),
            pl.BlockSpec((1, H2), lambda i: (0, 0)),
            pl.BlockSpec((1, 1), lambda i: (0, 0)),
        ],
        out_specs=pl.BlockSpec((BLK,), lambda i: (i,)),
        out_shape=jax.ShapeDtypeStruct((B,), jnp.float32),
    )(dense4, lin, w1, b1, w2, b2, wl, fmb)


def kernel(x, emb, fm_w, fm_b, w1, b1, w2, b2, w_last):
    offsets = (jnp.arange(F, dtype=jnp.int32) * V)
    x_off = x + offsets[None, :]
    idx = x_off.reshape(NW, ROWS_W, F).transpose(0, 2, 1).reshape(BF)
    emb_flat = emb.reshape(F * V, D)
    fmw_flat = fm_w.reshape(F * V)
    dense4, lin = _make_sc_gather()(idx, emb_flat, fmw_flat)
    w1p = jnp.pad(w1, ((0, 0), (0, NRS * 128 - F * D)))
    return _tc_forward(dense4, lin, w1p, b1.reshape(1, H1), w2,
                       b2.reshape(1, H2), w_last, fm_b.reshape(1, 1))


# speedup vs baseline: 45.5098x; 1.0047x over previous
import functools

import jax
import jax.numpy as jnp
from jax import lax
from jax.experimental import pallas as pl
from jax.experimental.pallas import tpu as pltpu
from jax.experimental.pallas import tpu_sc as plsc

B, F, V, D = 4096, 26, 1000, 16
H1, H2 = 400, 400
NC, NS = 2, 16
NW = NC * NS
BF = B * F
PER_W = BF // NW
ROWS_W = B // NW
NRS = 4


@functools.cache
def _make_sc_gather():
    mesh = plsc.VectorSubcoreMesh(
        core_axis_name="c", subcore_axis_name="s",
        num_cores=NC, num_subcores=NS)

    @functools.partial(
        pl.kernel,
        mesh=mesh,
        out_type=(
            jax.ShapeDtypeStruct((NRS, B, 128), jnp.float32),
            jax.ShapeDtypeStruct((B,), jnp.float32),
        ),
        scratch_types=[
            pltpu.VMEM((PER_W,), jnp.int32),
            pltpu.VMEM((PER_W, D), jnp.float32),
            pltpu.VMEM((ROWS_W,), jnp.float32),
            pltpu.VMEM((F * V,), jnp.float32),
            pltpu.SemaphoreType.DMA,
            pltpu.SemaphoreType.DMA,
        ],
        compiler_params=pltpu.CompilerParams(
            use_tc_tiling_on_sc=False, needs_layout_passes=False),
    )
    def sc_gather(idx_hbm, emb_hbm, fmw_hbm, dense_out, lin_out,
                  idx_v, rows_v, lin_v, fmtab_v, sem_e, sem_w):
        wid = lax.axis_index("s") * NC + lax.axis_index("c")
        base = wid * PER_W
        brow = wid * ROWS_W
        pltpu.sync_copy(idx_hbm.at[pl.ds(base, PER_W)], idx_v)
        cp_e = pltpu.async_copy(emb_hbm.at[idx_v], rows_v, sem_e)
        pltpu.sync_copy(fmw_hbm, fmtab_v)

        def fm_body(g, _):
            acc = jnp.zeros((16,), jnp.float32)
            for f in range(F):
                ii = idx_v[pl.ds(f * ROWS_W + g * 16, 16)]
                acc = acc + plsc.load_gather(fmtab_v, [ii])
            lin_v[pl.ds(g * 16, 16)] = acc
            return 0

        lax.fori_loop(0, ROWS_W // 16, fm_body, 0)
        cp_e.wait()
        cps = []
        for f in range(F):
            rs, c0 = (f * D) // 128, (f * D) % 128
            cps.append(pltpu.async_copy(
                rows_v.at[pl.ds(f * ROWS_W, ROWS_W)],
                dense_out.at[rs, pl.ds(brow, ROWS_W), pl.ds(c0, D)],
                sem_w))
        for cp in cps:
            cp.wait()
        pltpu.sync_copy(lin_v, lin_out.at[pl.ds(brow, ROWS_W)])

    return sc_gather


BLK = 1024


def _tc_body(dense_ref, lin_ref, w1_ref, b1_ref, w2_ref, b2_ref, wl_ref,
             fmb_ref, out_ref):
    d4 = [dense_ref[rs] for rs in range(NRS)]
    padcol = lax.broadcasted_iota(jnp.int32, (BLK, 128), 1)
    d4[3] = jnp.where(padcol < (F * D - 3 * 128), d4[3], 0.0)
    r = lax.broadcasted_iota(jnp.int32, (128, D), 0)
    c = lax.broadcasted_iota(jnp.int32, (128, D), 1)
    msel = jnp.where((r % D) == c, 1.0, 0.0).astype(jnp.float32)
    dsum = d4[0] + d4[1] + d4[2] + d4[3]
    dsq = d4[0] * d4[0] + d4[1] * d4[1] + d4[2] * d4[2] + d4[3] * d4[3]
    s = jnp.dot(dsum, msel, preferred_element_type=jnp.float32)
    ss = jnp.dot(dsq, msel, preferred_element_type=jnp.float32)
    inter = 0.5 * jnp.sum(s * s - ss, axis=1)
    dn = (((1,), (1,)), ((), ()))
    a1 = b1_ref[...]
    for rs in range(NRS):
        a1 = a1 + lax.dot_general(d4[rs].astype(jnp.bfloat16),
                                  w1_ref[:, pl.ds(rs * 128, 128)
                                         ].astype(jnp.bfloat16),
                                  dn, preferred_element_type=jnp.float32)
    a1 = jnp.maximum(a1, 0.0)
    a2 = lax.dot_general(a1.astype(jnp.bfloat16),
                         w2_ref[...].astype(jnp.bfloat16), dn,
                         preferred_element_type=jnp.float32) + b2_ref[...]
    a2 = jnp.maximum(a2, 0.0)
    y = lax.dot_general(a2, wl_ref[...], dn,
                        preferred_element_type=jnp.float32)
    out_ref[...] = lin_ref[...] + inter + y[:, 0] + fmb_ref[0, 0]


def _tc_forward(dense4, lin, w1, b1, w2, b2, wl, fmb):
    return pl.pallas_call(
        _tc_body,
        grid=(B // BLK,),
        in_specs=[
            pl.BlockSpec((NRS, BLK, 128), lambda i: (0, i, 0)),
            pl.BlockSpec((BLK,), lambda i: (i,)),
            pl.BlockSpec((H1, NRS * 128), lambda i: (0, 0)),
            pl.BlockSpec((1, H1), lambda i: (0, 0)),
            pl.BlockSpec((H2, H1), lambda i: (0, 0)),
            pl.BlockSpec((1, H2), lambda i: (0, 0)),
            pl.BlockSpec((1, H2), lambda i: (0, 0)),
            pl.BlockSpec((1, 1), lambda i: (0, 0)),
        ],
        out_specs=pl.BlockSpec((BLK,), lambda i: (i,)),
        out_shape=jax.ShapeDtypeStruct((B,), jnp.float32),
    )(dense4, lin, w1, b1, w2, b2, wl, fmb)


def kernel(x, emb, fm_w, fm_b, w1, b1, w2, b2, w_last):
    offsets = (jnp.arange(F, dtype=jnp.int32) * V)
    x_off = x + offsets[None, :]
    idx = x_off.reshape(NW, ROWS_W, F).transpose(0, 2, 1).reshape(BF)
    emb_flat = emb.reshape(F * V, D)
    fmw_flat = fm_w.reshape(F * V)
    dense4, lin = _make_sc_gather()(idx, emb_flat, fmw_flat)
    w1p = jnp.pad(w1, ((0, 0), (0, NRS * 128 - F * D)))
    return _tc_forward(dense4, lin, w1p, b1.reshape(1, H1), w2,
                       b2.reshape(1, H2), w_last, fm_b.reshape(1, 1))
